# baseline (speedup 1.0000x reference)
import functools

import jax
import jax.numpy as jnp
from jax import lax
from jax.experimental import pallas as pl
from jax.experimental.pallas import tpu as pltpu
from jax.experimental.pallas import tpu_sc as plsc

_CHUNK_ROWS = 32


def _make_sc_copy(seq_len, d_model, dtype):
    info = plsc.get_sparse_core_info()
    nc, ns = info.num_cores, info.num_subcores
    nw = nc * ns
    rows_per = seq_len // nw
    npairs = rows_per // (2 * _CHUNK_ROWS)
    mesh = plsc.VectorSubcoreMesh(core_axis_name="c", subcore_axis_name="s")

    @functools.partial(
        pl.kernel,
        mesh=mesh,
        out_type=jax.ShapeDtypeStruct((seq_len, d_model), dtype),
        scratch_types=[
            pltpu.VMEM((_CHUNK_ROWS, d_model), dtype),
            pltpu.VMEM((_CHUNK_ROWS, d_model), dtype),
            pltpu.SemaphoreType.DMA,
            pltpu.SemaphoreType.DMA,
            pltpu.SemaphoreType.DMA,
            pltpu.SemaphoreType.DMA,
        ],
    )
    def sc_copy(table_hbm, out_hbm, buf0, buf1, g0, g1, s0, s1):
        wid = lax.axis_index("c") * ns + lax.axis_index("s")
        base = wid * rows_per

        def drain_scatters():
            pltpu.make_async_copy(
                buf0, out_hbm.at[pl.ds(base, _CHUNK_ROWS)], s0
            ).wait()
            pltpu.make_async_copy(
                buf1, out_hbm.at[pl.ds(base, _CHUNK_ROWS)], s1
            ).wait()

        def body(i, carry):
            lo = base + i * 2 * _CHUNK_ROWS

            @pl.when(i > 0)
            def _():
                drain_scatters()

            ga = pltpu.async_copy(
                table_hbm.at[pl.ds(lo, _CHUNK_ROWS)], buf0, g0
            )
            ga.wait()
            pltpu.async_copy(
                buf0, out_hbm.at[pl.ds(lo, _CHUNK_ROWS)], s0
            )
            gb = pltpu.async_copy(
                table_hbm.at[pl.ds(lo + _CHUNK_ROWS, _CHUNK_ROWS)], buf1, g1
            )
            gb.wait()
            pltpu.async_copy(
                buf1, out_hbm.at[pl.ds(lo + _CHUNK_ROWS, _CHUNK_ROWS)], s1
            )
            return carry

        lax.fori_loop(0, npairs, body, 0)
        drain_scatters()

    return sc_copy


def kernel(x, pos_table):
    seq_len = x.shape[1]
    d_model = pos_table.shape[1]
    table = pos_table[:seq_len]
    out = _make_sc_copy(seq_len, d_model, pos_table.dtype)(table)
    return out[None]


# speedup vs baseline: 1.0424x; 1.0424x over previous
import functools

import jax
import jax.numpy as jnp
from jax import lax
from jax.experimental import pallas as pl
from jax.experimental.pallas import tpu as pltpu
from jax.experimental.pallas import tpu_sc as plsc

_CHUNK_ROWS = 32
_NBUF = 3


def _make_sc_copy(seq_len, d_model, dtype):
    info = plsc.get_sparse_core_info()
    nc, ns = info.num_cores, info.num_subcores
    nw = nc * ns
    rows_per = seq_len // nw
    nchunks = rows_per // _CHUNK_ROWS
    mesh = plsc.VectorSubcoreMesh(core_axis_name="c", subcore_axis_name="s")

    scratch = [pltpu.VMEM((_CHUNK_ROWS, d_model), dtype)] * _NBUF
    scratch += [pltpu.SemaphoreType.DMA] * (2 * _NBUF)

    @functools.partial(
        pl.kernel,
        mesh=mesh,
        out_type=jax.ShapeDtypeStruct((seq_len, d_model), dtype),
        scratch_types=scratch,
    )
    def sc_copy(table_hbm, out_hbm, *scr):
        bufs = scr[:_NBUF]
        gsem = scr[_NBUF:2 * _NBUF]
        ssem = scr[2 * _NBUF:]
        wid = lax.axis_index("c") * ns + lax.axis_index("s")
        base = wid * rows_per
        scat = [None] * _NBUF
        for i in range(nchunks):
            b = i % _NBUF
            lo = base + i * _CHUNK_ROWS
            if scat[b] is not None:
                scat[b].wait()
            gath = pltpu.async_copy(
                table_hbm.at[pl.ds(lo, _CHUNK_ROWS)], bufs[b], gsem[b]
            )
            gath.wait()
            scat[b] = pltpu.async_copy(
                bufs[b], out_hbm.at[pl.ds(lo, _CHUNK_ROWS)], ssem[b]
            )
        for b in range(_NBUF):
            if scat[b] is not None:
                scat[b].wait()

    return sc_copy


def kernel(x, pos_table):
    seq_len = x.shape[1]
    d_model = pos_table.shape[1]
    table = pos_table[:seq_len]
    out = _make_sc_copy(seq_len, d_model, pos_table.dtype)(table)
    return out[None]
